# baseline (speedup 1.0000x reference)
import dataclasses

import jax
import jax.numpy as jnp
from jax import lax
from jax.experimental import pallas as pl
from jax.experimental.pallas import tpu as pltpu
from jax.experimental.pallas import tpu_sc as plsc

EMBED_DIM = 128
GAZ_DIM = 64
OUT_DIM = EMBED_DIM + GAZ_DIM
WINDOW = 128
LANES = 16
NWORK = 32


def _embed_concat(sentence_data, gazetteers_data, word_table):
    num_tokens = sentence_data.shape[0]
    nw = num_tokens // WINDOW
    pw = nw // NWORK
    idx3d = sentence_data.reshape(NWORK, pw, WINDOW)
    gaz_t4 = gazetteers_data.T.reshape(GAZ_DIM // 8, 1, 8, num_tokens)
    mesh = plsc.VectorSubcoreMesh(core_axis_name="core",
                                  subcore_axis_name="subcore")

    cp = pltpu.CompilerParams()
    if "needs_layout_passes" in pltpu.CompilerParams.__dataclass_fields__:
        cp = dataclasses.replace(cp, needs_layout_passes=False)

    @pl.kernel(
        out_type=jax.ShapeDtypeStruct((OUT_DIM // 8, nw, 8, WINDOW),
                                      jnp.float32),
        mesh=mesh,
        compiler_params=cp,
        scratch_types=[
            pltpu.VMEM((pw, WINDOW), jnp.int32),
            pltpu.VMEM((WINDOW, EMBED_DIM), jnp.float32),
            pltpu.VMEM((WINDOW, EMBED_DIM), jnp.float32),
            pltpu.VMEM((WINDOW, EMBED_DIM), jnp.float32),
            pltpu.VMEM((WINDOW, EMBED_DIM), jnp.float32),
            pltpu.VMEM((OUT_DIM // 8, 1, 8, WINDOW), jnp.float32),
            pltpu.VMEM((OUT_DIM // 8, 1, 8, WINDOW), jnp.float32),
            pltpu.SemaphoreType.DMA,
            pltpu.SemaphoreType.DMA,
            pltpu.SemaphoreType.DMA,
            pltpu.SemaphoreType.DMA,
            pltpu.SemaphoreType.DMA,
            pltpu.SemaphoreType.DMA,
            pltpu.SemaphoreType.DMA,
            pltpu.SemaphoreType.DMA,
            pltpu.SemaphoreType.DMA,
        ],
    )
    def kern(idx_hbm, gazt_hbm, table_hbm, out_hbm,
             idx_all, scr_a, scr_b, scr_c, scr_d, ob_a, ob_b,
             isem, gsem_a, gsem_b, gsem_c, gsem_d,
             zsem_a, zsem_b, osem_a, osem_b):
        wid = lax.axis_index("subcore") * 2 + lax.axis_index("core")
        base = wid * pw

        pltpu.async_copy(idx_hbm.at[wid], idx_all, isem).wait()
        pltpu.async_copy(table_hbm.at[idx_all.at[0]], scr_a, gsem_a)
        pltpu.async_copy(table_hbm.at[idx_all.at[1]], scr_b, gsem_b)
        pltpu.async_copy(table_hbm.at[idx_all.at[2]], scr_c, gsem_c)
        pltpu.async_copy(table_hbm.at[idx_all.at[3]], scr_d, gsem_d)

        tok = lax.iota(jnp.int32, LANES)
        zero = jnp.zeros((LANES,), jnp.int32)
        rpat = [tok * EMBED_DIM + ((tok + d) & (LANES - 1))
                for d in range(LANES)]
        wpat = [((tok + d) & (LANES - 1)) * WINDOW + tok
                for d in range(LANES)]

        def out_dst(j):
            return out_hbm.at[pl.ds(0, OUT_DIM // 8), pl.ds(base + j, 1)]

        def stage(j, scr, ob, gsem, zsem, osem):
            @pl.when(j >= 2)
            def _():
                pltpu.make_async_copy(ob, out_dst(0), osem).wait()

            pltpu.async_copy(
                gazt_hbm.at[:, :, :, pl.ds((base + j) * WINDOW, WINDOW)],
                ob.at[pl.ds(EMBED_DIM // 8, GAZ_DIM // 8)], zsem)

            pltpu.make_async_copy(table_hbm.at[pl.ds(0, WINDOW)], scr,
                                  gsem).wait()

            nsub = (WINDOW // LANES) * (EMBED_DIM // LANES)

            @plsc.parallel_loop(0, nsub, unroll=4)
            def _(m):
                t0 = (m >> 3) * LANES
                f0 = (m & 7) * LANES
                for d in range(LANES):
                    vals = plsc.load_gather(
                        scr, [zero, rpat[d] + (t0 * EMBED_DIM + f0)])
                    plsc.store_scatter(
                        ob, [zero, zero, zero,
                             wpat[d] + (f0 * WINDOW + t0)], vals)

            @pl.when(j + 4 < pw)
            def _():
                pltpu.async_copy(table_hbm.at[idx_all.at[j + 4]], scr, gsem)

            pltpu.make_async_copy(
                gazt_hbm.at[:, :, :, pl.ds(0, WINDOW)],
                ob.at[pl.ds(EMBED_DIM // 8, GAZ_DIM // 8)], zsem).wait()
            pltpu.async_copy(ob, out_dst(j), osem)

        @pl.loop(0, pw - 2, step=4)
        def _(j):
            stage(j, scr_a, ob_a, gsem_a, zsem_a, osem_a)
            stage(j + 1, scr_b, ob_b, gsem_b, zsem_b, osem_b)
            stage(j + 2, scr_c, ob_a, gsem_c, zsem_a, osem_a)
            stage(j + 3, scr_d, ob_b, gsem_d, zsem_b, osem_b)

        stage(jnp.int32(pw - 2), scr_a, ob_a, gsem_a, zsem_a, osem_a)
        stage(jnp.int32(pw - 1), scr_b, ob_b, gsem_b, zsem_b, osem_b)

        pltpu.make_async_copy(ob_a, out_dst(0), osem_a).wait()
        pltpu.make_async_copy(ob_b, out_dst(0), osem_b).wait()

    out_tiled = kern(idx3d, gaz_t4, word_table)
    return out_tiled.transpose(1, 3, 0, 2).reshape(num_tokens, OUT_DIM)


def kernel(sentence_data, batch_sizes, gazetteers_data, word_table):
    out = _embed_concat(sentence_data, gazetteers_data, word_table)
    return out, batch_sizes


# speedup vs baseline: 1.0025x; 1.0025x over previous
import dataclasses

import jax
import jax.numpy as jnp
from jax import lax
from jax.experimental import pallas as pl
from jax.experimental.pallas import tpu as pltpu
from jax.experimental.pallas import tpu_sc as plsc

EMBED_DIM = 128
GAZ_DIM = 64
OUT_DIM = EMBED_DIM + GAZ_DIM
WINDOW = 128
LANES = 16
NWORK = 32


def _embed_concat(sentence_data, gazetteers_data, word_table):
    num_tokens = sentence_data.shape[0]
    nw = num_tokens // WINDOW
    pw = nw // NWORK
    idx3d = sentence_data.reshape(NWORK, pw, WINDOW)
    gaz_t4 = gazetteers_data.T.reshape(GAZ_DIM // 8, 1, 8, num_tokens)
    mesh = plsc.VectorSubcoreMesh(core_axis_name="core",
                                  subcore_axis_name="subcore")

    cp = pltpu.CompilerParams()
    if "needs_layout_passes" in pltpu.CompilerParams.__dataclass_fields__:
        cp = dataclasses.replace(cp, needs_layout_passes=False)

    @pl.kernel(
        out_type=jax.ShapeDtypeStruct((OUT_DIM // 8, nw, 8, WINDOW),
                                      jnp.float32),
        mesh=mesh,
        compiler_params=cp,
        scratch_types=[
            pltpu.VMEM((pw, WINDOW), jnp.int32),
            pltpu.VMEM((WINDOW, EMBED_DIM), jnp.float32),
            pltpu.VMEM((WINDOW, EMBED_DIM), jnp.float32),
            pltpu.VMEM((WINDOW, EMBED_DIM), jnp.float32),
            pltpu.VMEM((WINDOW, EMBED_DIM), jnp.float32),
            pltpu.VMEM((OUT_DIM // 8, 1, 8, WINDOW), jnp.float32),
            pltpu.VMEM((OUT_DIM // 8, 1, 8, WINDOW), jnp.float32),
            pltpu.SemaphoreType.DMA,
            pltpu.SemaphoreType.DMA,
            pltpu.SemaphoreType.DMA,
            pltpu.SemaphoreType.DMA,
            pltpu.SemaphoreType.DMA,
            pltpu.SemaphoreType.DMA,
            pltpu.SemaphoreType.DMA,
            pltpu.SemaphoreType.DMA,
            pltpu.SemaphoreType.DMA,
        ],
    )
    def kern(idx_hbm, gazt_hbm, table_hbm, out_hbm,
             idx_all, scr_a, scr_b, scr_c, scr_d, ob_a, ob_b,
             isem, gsem_a, gsem_b, gsem_c, gsem_d,
             zsem_a, zsem_b, osem_a, osem_b):
        wid = lax.axis_index("subcore") * 2 + lax.axis_index("core")
        base = wid * pw

        pltpu.async_copy(idx_hbm.at[wid], idx_all, isem).wait()
        pltpu.async_copy(table_hbm.at[idx_all.at[0]], scr_a, gsem_a)
        pltpu.async_copy(table_hbm.at[idx_all.at[1]], scr_b, gsem_b)
        pltpu.async_copy(table_hbm.at[idx_all.at[2]], scr_c, gsem_c)
        pltpu.async_copy(table_hbm.at[idx_all.at[3]], scr_d, gsem_d)

        tok = lax.iota(jnp.int32, LANES)
        zero = jnp.zeros((LANES,), jnp.int32)
        rpat = [tok * EMBED_DIM + ((tok + d) & (LANES - 1))
                for d in range(LANES)]
        wpat = [((tok + d) & (LANES - 1)) * WINDOW + tok
                for d in range(LANES)]

        def out_dst(j):
            return out_hbm.at[pl.ds(0, OUT_DIM // 8), pl.ds(base + j, 1)]

        def stage(j, scr, ob, gsem, zsem, osem):
            @pl.when(j >= 2)
            def _():
                pltpu.make_async_copy(ob, out_dst(0), osem).wait()

            pltpu.async_copy(
                gazt_hbm.at[:, :, :, pl.ds((base + j) * WINDOW, WINDOW)],
                ob.at[pl.ds(EMBED_DIM // 8, GAZ_DIM // 8)], zsem)

            pltpu.make_async_copy(table_hbm.at[pl.ds(0, WINDOW)], scr,
                                  gsem).wait()

            nsub = (WINDOW // LANES) * (EMBED_DIM // LANES)

            @plsc.parallel_loop(0, nsub, unroll=2)
            def _(m):
                t0 = (m >> 3) * LANES
                f0 = (m & 7) * LANES
                for d in range(LANES):
                    vals = plsc.load_gather(
                        scr, [zero, rpat[d] + (t0 * EMBED_DIM + f0)])
                    plsc.store_scatter(
                        ob, [zero, zero, zero,
                             wpat[d] + (f0 * WINDOW + t0)], vals)

            @pl.when(j + 4 < pw)
            def _():
                pltpu.async_copy(table_hbm.at[idx_all.at[j + 4]], scr, gsem)

            pltpu.make_async_copy(
                gazt_hbm.at[:, :, :, pl.ds(0, WINDOW)],
                ob.at[pl.ds(EMBED_DIM // 8, GAZ_DIM // 8)], zsem).wait()
            pltpu.async_copy(ob, out_dst(j), osem)

        @pl.loop(0, pw - 2, step=4)
        def _(j):
            stage(j, scr_a, ob_a, gsem_a, zsem_a, osem_a)
            stage(j + 1, scr_b, ob_b, gsem_b, zsem_b, osem_b)
            stage(j + 2, scr_c, ob_a, gsem_c, zsem_a, osem_a)
            stage(j + 3, scr_d, ob_b, gsem_d, zsem_b, osem_b)

        stage(jnp.int32(pw - 2), scr_a, ob_a, gsem_a, zsem_a, osem_a)
        stage(jnp.int32(pw - 1), scr_b, ob_b, gsem_b, zsem_b, osem_b)

        pltpu.make_async_copy(ob_a, out_dst(0), osem_a).wait()
        pltpu.make_async_copy(ob_b, out_dst(0), osem_b).wait()

    out_tiled = kern(idx3d, gaz_t4, word_table)
    return out_tiled.transpose(1, 3, 0, 2).reshape(num_tokens, OUT_DIM)


def kernel(sentence_data, batch_sizes, gazetteers_data, word_table):
    out = _embed_concat(sentence_data, gazetteers_data, word_table)
    return out, batch_sizes
